# baseline (speedup 1.0000x reference)
import functools

import jax
import jax.numpy as jnp
from jax import lax
from jax.experimental import pallas as pl
from jax.experimental.pallas import tpu as pltpu
from jax.experimental.pallas import tpu_sc as plsc

NN = 24
NE = 512
LANES = 16
CM = NN * NN
NPLANES = 2 * LANES
ACC = NPLANES * CM

_HIGH = jax.lax.Precision.HIGHEST


def _sc_edge_counts(edge_index, zeros):
    mesh = plsc.VectorSubcoreMesh(core_axis_name="c", subcore_axis_name="s",
                                  num_cores=1, num_subcores=16)

    @functools.partial(
        pl.kernel,
        mesh=mesh,
        compiler_params=pltpu.CompilerParams(needs_layout_passes=False),
        out_type=jax.ShapeDtypeStruct((2, NN * 128), jnp.float32),
        scratch_types=[
            pltpu.VMEM((2, NE), jnp.int32),
            pltpu.VMEM((ACC,), jnp.float32),
            pltpu.VMEM((NN * 128,), jnp.float32),
            pltpu.SemaphoreType.DMA,
            pltpu.SemaphoreType.DMA,
            pltpu.SemaphoreType.DMA,
        ],
    )
    def k(ei_hbm, z_hbm, out_hbm, ei_v, acc_v, red_v, sem1, sem2, sem3):
        sid = lax.axis_index("s")

        @pl.when(sid < 2)
        def _():
            cp1 = pltpu.async_copy(ei_hbm, ei_v, sem1)
            cp2 = pltpu.async_copy(z_hbm, acc_v.at[pl.ds(0, LANES * CM)],
                                   sem2)
            cp3 = pltpu.async_copy(
                z_hbm, acc_v.at[pl.ds(LANES * CM, LANES * CM)], sem3)
            cp1.wait()
            cp2.wait()
            cp3.wait()
            lane = lax.iota(jnp.int32, LANES) * CM
            ones = jnp.ones((LANES,), jnp.float32)
            ebase = sid * (NE // 2)

            def scat_body(j, carry):
                off = ebase + j * (2 * LANES)
                s0 = ei_v[0, pl.ds(off, LANES)]
                d0 = ei_v[1, pl.ds(off, LANES)]
                plsc.addupdate_scatter(acc_v, [lane + d0 * NN + s0], ones)
                s1 = ei_v[0, pl.ds(off + LANES, LANES)]
                d1 = ei_v[1, pl.ds(off + LANES, LANES)]
                plsc.addupdate_scatter(
                    acc_v, [lane + (LANES * CM) + d1 * NN + s1], ones)
                return carry

            lax.fori_loop(0, NE // (4 * LANES), scat_body, 0)

            def red_body(d, carry):
                for c0 in (0, 8):
                    v = acc_v[pl.ds(d * NN + c0, LANES)]
                    for l in range(1, NPLANES):
                        v = v + acc_v[pl.ds(l * CM + d * NN + c0, LANES)]
                    red_v[pl.ds(d * 128 + c0, LANES)] = v
                return carry

            lax.fori_loop(0, NN, red_body, 0)
            pltpu.sync_copy(red_v, out_hbm.at[sid])

    return k(edge_index, zeros)


def _dense_body(cp_ref, x_ref,
                w1, b1, w2, b2, w3, b3, w4, b4, w5, b5, w6, b6, w7, b7,
                fc1_ref, fb1_ref, fc2_ref, fb2_ref, o_ref):
    C = cp_ref[:].sum(axis=0)[:, :NN]
    deg = C.sum(axis=1, keepdims=True) + 1.0
    u = 1.0 / jnp.sqrt(deg)
    r = lax.broadcasted_iota(jnp.int32, (NN, NN), 0)
    c = lax.broadcasted_iota(jnp.int32, (NN, NN), 1)
    P = C + (r == c).astype(jnp.float32)

    h = x_ref[:]
    for w_ref, b_ref in ((w1, b1), (w2, b2), (w3, b3), (w4, b4),
                         (w5, b5), (w6, b6), (w7, b7)):
        g = jnp.dot(h, w_ref[:], precision=_HIGH,
                    preferred_element_type=jnp.float32)
        g = g * u
        m = jnp.dot(P, g, precision=_HIGH,
                    preferred_element_type=jnp.float32)
        h = jnp.maximum(m * u + b_ref[:].reshape(1, -1), 0.0)

    t = h[:, :, None] * fc1_ref[:].reshape(NN, 32, 128)
    z = t.sum(axis=0).sum(axis=0, keepdims=True) \
        + fb1_ref[:].reshape(1, -1)
    z2 = jnp.dot(z, fc2_ref[:], precision=_HIGH,
                 preferred_element_type=jnp.float32) \
        + fb2_ref[:].reshape(1, -1)
    mx = jnp.max(z2, axis=1, keepdims=True)
    e = jnp.exp(z2 - mx)
    o_ref[:] = (z2 - mx) - jnp.log(jnp.sum(e, axis=1, keepdims=True))


def kernel(x, edge_index, W1, b1, W2, b2, W3, b3, W4, b4, W5, b5, W6, b6,
           W7, b7, fcW1, fcb1, fcW2, fcb2):
    cp = _sc_edge_counts(
        edge_index.astype(jnp.int32),
        jnp.zeros((LANES * CM,), jnp.float32)).reshape(2, NN, 128)
    args = [cp, x,
            W1, b1, W2, b2, W3, b3, W4, b4, W5, b5, W6, b6, W7, b7,
            fcW1, fcb1, fcW2, fcb2]
    return pl.pallas_call(
        _dense_body,
        out_shape=jax.ShapeDtypeStruct((1, 2), jnp.float32),
    )(*args)


# speedup vs baseline: 1.0688x; 1.0688x over previous
import functools

import jax
import jax.numpy as jnp
from jax import lax
from jax.experimental import pallas as pl
from jax.experimental.pallas import tpu as pltpu
from jax.experimental.pallas import tpu_sc as plsc

NN = 24
NE = 512
LANES = 16
CM = NN * NN
NPLANES = 2 * LANES
ACC = NPLANES * CM

_HIGH = jax.lax.Precision.HIGHEST


def _sc_edge_counts(edge_index, zeros):
    mesh = plsc.VectorSubcoreMesh(core_axis_name="c", subcore_axis_name="s",
                                  num_cores=1, num_subcores=16)

    @functools.partial(
        pl.kernel,
        mesh=mesh,
        compiler_params=pltpu.CompilerParams(needs_layout_passes=False),
        out_type=jax.ShapeDtypeStruct((NN * 128,), jnp.float32),
        scratch_types=[
            pltpu.VMEM((2, NE), jnp.int32),
            pltpu.VMEM((ACC,), jnp.float32),
            pltpu.VMEM((NN * 128,), jnp.float32),
            pltpu.SemaphoreType.DMA,
            pltpu.SemaphoreType.DMA,
            pltpu.SemaphoreType.DMA,
        ],
    )
    def k(ei_hbm, z_hbm, out_hbm, ei_v, acc_v, red_v, sem1, sem2, sem3):
        @pl.when(lax.axis_index("s") == 0)
        def _():
            cp1 = pltpu.async_copy(ei_hbm, ei_v, sem1)
            cp2 = pltpu.async_copy(z_hbm, acc_v.at[pl.ds(0, LANES * CM)],
                                   sem2)
            cp3 = pltpu.async_copy(
                z_hbm, acc_v.at[pl.ds(LANES * CM, LANES * CM)], sem3)
            cp1.wait()
            cp2.wait()
            cp3.wait()
            lane = lax.iota(jnp.int32, LANES) * CM
            ones = jnp.ones((LANES,), jnp.float32)

            def scat_body(j, carry):
                off = j * (2 * LANES)
                s0 = ei_v[0, pl.ds(off, LANES)]
                d0 = ei_v[1, pl.ds(off, LANES)]
                plsc.addupdate_scatter(acc_v, [lane + d0 * NN + s0], ones)
                s1 = ei_v[0, pl.ds(off + LANES, LANES)]
                d1 = ei_v[1, pl.ds(off + LANES, LANES)]
                plsc.addupdate_scatter(
                    acc_v, [lane + (LANES * CM) + d1 * NN + s1], ones)
                return carry

            lax.fori_loop(0, NE // (2 * LANES), scat_body, 0)

            def red_body(d, carry):
                for c0 in (0, 8):
                    v = acc_v[pl.ds(d * NN + c0, LANES)]
                    for l in range(1, NPLANES):
                        v = v + acc_v[pl.ds(l * CM + d * NN + c0, LANES)]
                    red_v[pl.ds(d * 128 + c0, LANES)] = v
                return carry

            lax.fori_loop(0, NN, red_body, 0)
            pltpu.sync_copy(red_v, out_hbm)

    return k(edge_index, zeros)


def _dense_body(cp_ref, x_ref,
                w1, b1, w2, b2, w3, b3, w4, b4, w5, b5, w6, b6, w7, b7,
                fc1_ref, fb1_ref, fc2_ref, fb2_ref, o_ref):
    C = cp_ref[:][:, :NN]
    deg = C.sum(axis=1, keepdims=True) + 1.0
    u = 1.0 / jnp.sqrt(deg)
    r = lax.broadcasted_iota(jnp.int32, (NN, NN), 0)
    c = lax.broadcasted_iota(jnp.int32, (NN, NN), 1)
    P = C + (r == c).astype(jnp.float32)

    h = x_ref[:]
    for w_ref, b_ref in ((w1, b1), (w2, b2), (w3, b3), (w4, b4),
                         (w5, b5), (w6, b6), (w7, b7)):
        g = jnp.dot(h, w_ref[:], precision=_HIGH,
                    preferred_element_type=jnp.float32)
        g = g * u
        m = jnp.dot(P, g, precision=_HIGH,
                    preferred_element_type=jnp.float32)
        h = jnp.maximum(m * u + b_ref[:].reshape(1, -1), 0.0)

    t = h[:, :, None] * fc1_ref[:].reshape(NN, 32, 128)
    z = t.sum(axis=0).sum(axis=0, keepdims=True) \
        + fb1_ref[:].reshape(1, -1)
    z2 = jnp.dot(z, fc2_ref[:], precision=_HIGH,
                 preferred_element_type=jnp.float32) \
        + fb2_ref[:].reshape(1, -1)
    mx = jnp.max(z2, axis=1, keepdims=True)
    e = jnp.exp(z2 - mx)
    o_ref[:] = (z2 - mx) - jnp.log(jnp.sum(e, axis=1, keepdims=True))


def kernel(x, edge_index, W1, b1, W2, b2, W3, b3, W4, b4, W5, b5, W6, b6,
           W7, b7, fcW1, fcb1, fcW2, fcb2):
    cp = _sc_edge_counts(
        edge_index.astype(jnp.int32),
        jnp.zeros((LANES * CM,), jnp.float32)).reshape(NN, 128)
    args = [cp, x,
            W1, b1, W2, b2, W3, b3, W4, b4, W5, b5, W6, b6, W7, b7,
            fcW1, fcb1, fcW2, fcb2]
    return pl.pallas_call(
        _dense_body,
        out_shape=jax.ShapeDtypeStruct((1, 2), jnp.float32),
    )(*args)


# speedup vs baseline: 1.1229x; 1.0506x over previous
import functools

import jax
import jax.numpy as jnp
from jax import lax
from jax.experimental import pallas as pl
from jax.experimental.pallas import tpu as pltpu
from jax.experimental.pallas import tpu_sc as plsc

NN = 24
NE = 512
LANES = 16
CM = NN * NN
ACC = LANES * CM

_HIGH = jax.lax.Precision.HIGHEST


def _sc_edge_counts(edge_index, zeros):
    mesh = plsc.VectorSubcoreMesh(core_axis_name="c", subcore_axis_name="s",
                                  num_cores=1, num_subcores=16)

    @functools.partial(
        pl.kernel,
        mesh=mesh,
        compiler_params=pltpu.CompilerParams(needs_layout_passes=False),
        out_type=jax.ShapeDtypeStruct((NN * 128,), jnp.float32),
        scratch_types=[
            pltpu.VMEM((2, NE), jnp.int32),
            pltpu.VMEM((ACC,), jnp.float32),
            pltpu.VMEM((NN * 128,), jnp.float32),
            pltpu.SemaphoreType.DMA,
            pltpu.SemaphoreType.DMA,
        ],
    )
    def k(ei_hbm, z_hbm, out_hbm, ei_v, acc_v, red_v, sem1, sem2):
        @pl.when(lax.axis_index("s") == 0)
        def _():
            cp1 = pltpu.async_copy(ei_hbm, ei_v, sem1)
            cp2 = pltpu.async_copy(z_hbm, acc_v, sem2)
            cp1.wait()
            cp2.wait()
            lane = lax.iota(jnp.int32, LANES) * CM
            ones = jnp.ones((LANES,), jnp.float32)

            def scat_body(j, carry):
                s = ei_v[0, pl.ds(j * LANES, LANES)]
                d = ei_v[1, pl.ds(j * LANES, LANES)]
                plsc.addupdate_scatter(acc_v, [lane + d * NN + s], ones)
                return carry

            lax.fori_loop(0, NE // LANES, scat_body, 0)

            def red_body(d, carry):
                for c0 in (0, 8):
                    v = acc_v[pl.ds(d * NN + c0, LANES)]
                    for l in range(1, LANES):
                        v = v + acc_v[pl.ds(l * CM + d * NN + c0, LANES)]
                    red_v[pl.ds(d * 128 + c0, LANES)] = v
                return carry

            lax.fori_loop(0, NN, red_body, 0)
            pltpu.sync_copy(red_v, out_hbm)

    return k(edge_index, zeros)


def _dense_body(cp_ref, x_ref,
                w1, b1, w2, b2, w3, b3, w4, b4, w5, b5, w6, b6, w7, b7,
                fc1_ref, fb1_ref, fc2_ref, fb2_ref, o_ref):
    C = cp_ref[:][:, :NN]
    deg = C.sum(axis=1, keepdims=True) + 1.0
    u = 1.0 / jnp.sqrt(deg)
    r = lax.broadcasted_iota(jnp.int32, (NN, NN), 0)
    c = lax.broadcasted_iota(jnp.int32, (NN, NN), 1)
    P = C + (r == c).astype(jnp.float32)

    h = x_ref[:]
    for w_ref, b_ref in ((w1, b1), (w2, b2), (w3, b3), (w4, b4),
                         (w5, b5), (w6, b6), (w7, b7)):
        g = jnp.dot(h, w_ref[:], precision=_HIGH,
                    preferred_element_type=jnp.float32)
        g = g * u
        m = jnp.dot(P, g, precision=_HIGH,
                    preferred_element_type=jnp.float32)
        h = jnp.maximum(m * u + b_ref[:].reshape(1, -1), 0.0)

    t = h[:, :, None] * fc1_ref[:].reshape(NN, 32, 128)
    z = t.sum(axis=0).sum(axis=0, keepdims=True) \
        + fb1_ref[:].reshape(1, -1)
    z2 = jnp.dot(z, fc2_ref[:], precision=_HIGH,
                 preferred_element_type=jnp.float32) \
        + fb2_ref[:].reshape(1, -1)
    mx = jnp.max(z2, axis=1, keepdims=True)
    e = jnp.exp(z2 - mx)
    o_ref[:] = (z2 - mx) - jnp.log(jnp.sum(e, axis=1, keepdims=True))


def kernel(x, edge_index, W1, b1, W2, b2, W3, b3, W4, b4, W5, b5, W6, b6,
           W7, b7, fcW1, fcb1, fcW2, fcb2):
    cp = _sc_edge_counts(edge_index.astype(jnp.int32),
                         jnp.zeros((ACC,), jnp.float32)).reshape(NN, 128)
    args = [cp, x,
            W1, b1, W2, b2, W3, b3, W4, b4, W5, b5, W6, b6, W7, b7,
            fcW1, fcb1, fcW2, fcb2]
    return pl.pallas_call(
        _dense_body,
        out_shape=jax.ShapeDtypeStruct((1, 2), jnp.float32),
    )(*args)
